# baseline (speedup 1.0000x reference)
import functools

import jax
import jax.numpy as jnp
from jax import lax
from jax.experimental import pallas as pl
from jax.experimental.pallas import tpu as pltpu
from jax.experimental.pallas import tpu_sc as plsc

_N, _L = 200000, 25
_NW = 32
_CH = 2432
_NSC = 199936
_NFULL = _NSC // _CH
_TAIL = _NSC - _NFULL * _CH
_NCHUNK = _NFULL + 1
_KPW = -(-_NCHUNK // _NW)
_ITEMS = [(s, k) for s in range(3) for k in range(_KPW)]

_mesh = plsc.VectorSubcoreMesh(core_axis_name="c", subcore_axis_name="s")


def _sc_body(xt, et, bef, aft, out, mm,
             xv, ev, ov, befv, aftv, accv, sx, se, so):
    wid = lax.axis_index("s") * 2 + lax.axis_index("c")
    pltpu.sync_copy(bef, befv)
    pltpu.sync_copy(aft, aftv)

    def regions(s, k):
        c = wid + k * _NW
        return c, s * 8, c * _CH

    def issue_in(s, k, b):
        c, r0, cb = regions(s, k)

        @pl.when(c < _NFULL)
        def _main():
            pltpu.async_copy(xt.at[pl.ds(r0, 8), pl.ds(cb, _CH)],
                             xv.at[b], sx.at[b])
            pltpu.async_copy(et.at[pl.ds(r0, 8), pl.ds(cb, _CH)],
                             ev.at[b], se.at[b])

        @pl.when(c == _NFULL)
        def _tail():
            pltpu.async_copy(xt.at[pl.ds(r0, 8), pl.ds(cb, _TAIL)],
                             xv.at[b, :, pl.ds(0, _TAIL)], sx.at[b])
            pltpu.async_copy(et.at[pl.ds(r0, 8), pl.ds(cb, _TAIL)],
                             ev.at[b, :, pl.ds(0, _TAIL)], se.at[b])

    def wait_in(s, k, b):
        c, r0, cb = regions(s, k)

        @pl.when(c < _NFULL)
        def _main():
            pltpu.make_async_copy(xt.at[pl.ds(r0, 8), pl.ds(cb, _CH)],
                                  xv.at[b], sx.at[b]).wait()
            pltpu.make_async_copy(et.at[pl.ds(r0, 8), pl.ds(cb, _CH)],
                                  ev.at[b], se.at[b]).wait()

        @pl.when(c == _NFULL)
        def _tail():
            pltpu.make_async_copy(xt.at[pl.ds(r0, 8), pl.ds(cb, _TAIL)],
                                  xv.at[b, :, pl.ds(0, _TAIL)], sx.at[b]).wait()
            pltpu.make_async_copy(et.at[pl.ds(r0, 8), pl.ds(cb, _TAIL)],
                                  ev.at[b, :, pl.ds(0, _TAIL)], se.at[b]).wait()

    def issue_out(s, k, b):
        c, r0, cb = regions(s, k)

        @pl.when(c < _NFULL)
        def _main():
            pltpu.async_copy(ov.at[b], out.at[pl.ds(r0, 8), pl.ds(cb, _CH)],
                             so.at[b])

        @pl.when(c == _NFULL)
        def _tail():
            pltpu.async_copy(ov.at[b, :, pl.ds(0, _TAIL)],
                             out.at[pl.ds(r0, 8), pl.ds(cb, _TAIL)], so.at[b])

    def wait_out(s, k, b):
        c, r0, cb = regions(s, k)

        @pl.when(c < _NFULL)
        def _main():
            pltpu.make_async_copy(ov.at[b],
                                  out.at[pl.ds(r0, 8), pl.ds(cb, _CH)],
                                  so.at[b]).wait()

        @pl.when(c == _NFULL)
        def _tail():
            pltpu.make_async_copy(ov.at[b, :, pl.ds(0, _TAIL)],
                                  out.at[pl.ds(r0, 8), pl.ds(cb, _TAIL)],
                                  so.at[b]).wait()

    def compute(s, k, b):
        c, _, _ = regions(s, k)

        def body(j, acc):
            col = j * 16
            for r in range(8):
                x = xv[b, r, pl.ds(col, 16)]
                e = ev[b, r, pl.ds(col, 16)]
                new = jnp.where(x == befv[...], aftv[...], x)
                ov[b, r, pl.ds(col, 16)] = new
                acc = acc | (new ^ e)
            return acc

        @pl.when(c < _NFULL)
        def _main():
            accv[...] = lax.fori_loop(0, _CH // 16, body, accv[...],
                                      unroll=2)

        @pl.when(c == _NFULL)
        def _tail():
            accv[...] = lax.fori_loop(0, _TAIL // 16, body, accv[...],
                                      unroll=2)

    accv[...] = jnp.zeros((16,), jnp.int32)
    nit = len(_ITEMS)
    issue_in(*_ITEMS[0], 0)
    for idx, (s, k) in enumerate(_ITEMS):
        b = idx % 2
        c = wid + k * _NW
        if idx + 1 < nit:
            issue_in(*_ITEMS[idx + 1], 1 - b)
        if idx >= 2:
            sp, kp = _ITEMS[idx - 2]

            @pl.when(wid + kp * _NW < _NCHUNK)
            def _(sp=sp, kp=kp, b=b):
                wait_out(sp, kp, b)

        @pl.when(c < _NCHUNK)
        def _do(s=s, k=k, b=b):
            wait_in(s, k, b)
            compute(s, k, b)
            issue_out(s, k, b)

    for idx in (nit - 2, nit - 1):
        s, k = _ITEMS[idx]

        @pl.when(wid + k * _NW < _NCHUNK)
        def _(s=s, k=k, b=idx % 2):
            wait_out(s, k, b)

    pltpu.sync_copy(accv, mm.at[wid])


_sc_call = functools.partial(
    pl.kernel,
    mesh=_mesh,
    out_type=[
        jax.ShapeDtypeStruct((_L, _N), jnp.int32),
        jax.ShapeDtypeStruct((_NW, 16), jnp.int32),
    ],
    scratch_types=[
        pltpu.VMEM((2, 8, _CH), jnp.int32),
        pltpu.VMEM((2, 8, _CH), jnp.int32),
        pltpu.VMEM((2, 8, _CH), jnp.int32),
        pltpu.VMEM((16,), jnp.int32),
        pltpu.VMEM((16,), jnp.int32),
        pltpu.VMEM((16,), jnp.int32),
        pltpu.SemaphoreType.DMA((2,)),
        pltpu.SemaphoreType.DMA((2,)),
        pltpu.SemaphoreType.DMA((2,)),
    ],
    compiler_params=pltpu.CompilerParams(use_tc_tiling_on_sc=True),
)(_sc_body)


_FBW = 50176
_FCB = -(-_N // _FBW)
_FG = _FCB + 3


def _fix_idx(i):
    return (jnp.where(i < _FCB, 3, i - _FCB),
            jnp.where(i < _FCB, i, _FCB - 1))


def _fix_body(scal_ref, x_ref, e_ref, prev_ref, o_ref, mm_ref):
    i = pl.program_id(0)
    before = scal_ref[0]
    after = scal_ref[1]
    x = x_ref[...]
    new = jnp.where(x == before, after, x)
    o_ref[...] = new
    rb, cb = _fix_idx(i)
    row = rb * 8 + jax.lax.broadcasted_iota(jnp.int32, (8, _FBW), 0)
    col = cb * _FBW + jax.lax.broadcasted_iota(jnp.int32, (8, _FBW), 1)
    d = (new != e_ref[...]) & (row < _L) & (col < _N)
    mismatch = jnp.any(d).astype(jnp.int32)

    @pl.when(i == 0)
    def _init():
        mm_ref[0] = mismatch

    @pl.when(i > 0)
    def _acc():
        mm_ref[0] = mm_ref[0] | mismatch


def _fixup(prev, xt, et, scal):
    return pl.pallas_call(
        _fix_body,
        grid=(_FG,),
        in_specs=[
            pl.BlockSpec(memory_space=pltpu.SMEM),
            pl.BlockSpec((8, _FBW), _fix_idx),
            pl.BlockSpec((8, _FBW), _fix_idx),
            pl.BlockSpec(memory_space=pl.ANY),
        ],
        out_specs=[
            pl.BlockSpec((8, _FBW), _fix_idx),
            pl.BlockSpec(memory_space=pltpu.SMEM),
        ],
        out_shape=[
            jax.ShapeDtypeStruct((_L, _N), jnp.int32),
            jax.ShapeDtypeStruct((1,), jnp.int32),
        ],
        input_output_aliases={3: 0},
    )(scal, xt, et, prev)


def kernel(ids, end_ids, reward_base, before_id, after_id):
    bef = jnp.full((16,), before_id, jnp.int32)
    aft = jnp.full((16,), after_id, jnp.int32)
    scal = jnp.stack([jnp.asarray(before_id, jnp.int32),
                      jnp.asarray(after_id, jnp.int32)])
    xt = ids.T
    et = end_ids.T
    out_sc, mm = _sc_call(xt, et, bef, aft)
    out, fmm = _fixup(out_sc, xt, et, scal)
    done = jnp.logical_not(jnp.any(mm)) & (fmm[0] == 0)
    reward = jnp.where(done, reward_base[0], jnp.zeros((), jnp.float32))
    return out.T, done, reward


# speedup vs baseline: 1.7437x; 1.7437x over previous
import functools

import jax
import jax.numpy as jnp
from jax import lax
from jax.experimental import pallas as pl
from jax.experimental.pallas import tpu as pltpu
from jax.experimental.pallas import tpu_sc as plsc

_N, _L = 200000, 25
_NW = 32
_CH = 3200
_NSC = 199936
_NFULL = _NSC // _CH
_TAIL = _NSC - _NFULL * _CH
_NCHUNK = _NFULL + 1
_KPW = -(-_NCHUNK // _NW)
_ITEMS = [(s, k) for s in range(3) for k in range(_KPW)]

_mesh = plsc.VectorSubcoreMesh(core_axis_name="c", subcore_axis_name="s")


def _sc_body(xt, et, bef, aft, mm, xv, ev, befv, aftv, accv, sx, se):
    wid = lax.axis_index("s") * 2 + lax.axis_index("c")
    pltpu.sync_copy(bef, befv)
    pltpu.sync_copy(aft, aftv)

    def issue_in(s, k, b):
        c = wid + k * _NW
        r0 = s * 8
        cb = c * _CH

        @pl.when(c < _NFULL)
        def _main():
            pltpu.async_copy(xt.at[pl.ds(r0, 8), pl.ds(cb, _CH)],
                             xv.at[b], sx.at[b])
            pltpu.async_copy(et.at[pl.ds(r0, 8), pl.ds(cb, _CH)],
                             ev.at[b], se.at[b])

        @pl.when(c == _NFULL)
        def _tail():
            pltpu.async_copy(xt.at[pl.ds(r0, 8), pl.ds(cb, _TAIL)],
                             xv.at[b, :, pl.ds(0, _TAIL)], sx.at[b])
            pltpu.async_copy(et.at[pl.ds(r0, 8), pl.ds(cb, _TAIL)],
                             ev.at[b, :, pl.ds(0, _TAIL)], se.at[b])

    def wait_in(s, k, b):
        c = wid + k * _NW
        r0 = s * 8
        cb = c * _CH

        @pl.when(c < _NFULL)
        def _main():
            pltpu.make_async_copy(xt.at[pl.ds(r0, 8), pl.ds(cb, _CH)],
                                  xv.at[b], sx.at[b]).wait()
            pltpu.make_async_copy(et.at[pl.ds(r0, 8), pl.ds(cb, _CH)],
                                  ev.at[b], se.at[b]).wait()

        @pl.when(c == _NFULL)
        def _tail():
            pltpu.make_async_copy(xt.at[pl.ds(r0, 8), pl.ds(cb, _TAIL)],
                                  xv.at[b, :, pl.ds(0, _TAIL)], sx.at[b]).wait()
            pltpu.make_async_copy(et.at[pl.ds(r0, 8), pl.ds(cb, _TAIL)],
                                  ev.at[b, :, pl.ds(0, _TAIL)], se.at[b]).wait()

    def compute(s, k, b):
        c = wid + k * _NW

        def body(j, acc):
            col = j * 16
            for r in range(8):
                x = xv[b, r, pl.ds(col, 16)]
                e = ev[b, r, pl.ds(col, 16)]
                new = jnp.where(x == befv[...], aftv[...], x)
                acc = acc | (new ^ e)
            return acc

        @pl.when(c < _NFULL)
        def _main():
            accv[...] = lax.fori_loop(0, _CH // 16, body, accv[...],
                                      unroll=2)

        @pl.when(c == _NFULL)
        def _tail():
            accv[...] = lax.fori_loop(0, _TAIL // 16, body, accv[...],
                                      unroll=2)

    accv[...] = jnp.zeros((16,), jnp.int32)
    nit = len(_ITEMS)
    issue_in(*_ITEMS[0], 0)
    for idx, (s, k) in enumerate(_ITEMS):
        b = idx % 2
        c = wid + k * _NW
        if idx + 1 < nit:
            issue_in(*_ITEMS[idx + 1], 1 - b)

        @pl.when(c < _NCHUNK)
        def _do(s=s, k=k, b=b):
            wait_in(s, k, b)
            compute(s, k, b)

    pltpu.sync_copy(accv, mm.at[wid])


_sc_call = functools.partial(
    pl.kernel,
    mesh=_mesh,
    out_type=jax.ShapeDtypeStruct((_NW, 16), jnp.int32),
    scratch_types=[
        pltpu.VMEM((2, 8, _CH), jnp.int32),
        pltpu.VMEM((2, 8, _CH), jnp.int32),
        pltpu.VMEM((16,), jnp.int32),
        pltpu.VMEM((16,), jnp.int32),
        pltpu.VMEM((16,), jnp.int32),
        pltpu.SemaphoreType.DMA((2,)),
        pltpu.SemaphoreType.DMA((2,)),
    ],
    compiler_params=pltpu.CompilerParams(use_tc_tiling_on_sc=True),
)(_sc_body)


_BW = 33408
_GRID = (_N + _BW - 1) // _BW


def _tc_body(scal_ref, x_ref, out_ref):
    before = scal_ref[0]
    after = scal_ref[1]
    x = x_ref[...]
    out_ref[...] = jnp.where(x == before, after, x)


def _tc_main(xt, scal):
    return pl.pallas_call(
        _tc_body,
        grid=(_GRID,),
        in_specs=[
            pl.BlockSpec(memory_space=pltpu.SMEM),
            pl.BlockSpec((_L, _BW), lambda i: (0, i)),
        ],
        out_specs=pl.BlockSpec((_L, _BW), lambda i: (0, i)),
        out_shape=jax.ShapeDtypeStruct((_L, _N), jnp.int32),
    )(scal, xt)


_FBW = 50176
_FCB = -(-_N // _FBW)
_FG = _FCB + 3


def _fix_idx(i):
    return (jnp.where(i < _FCB, 3, i - _FCB),
            jnp.where(i < _FCB, i, _FCB - 1))


def _fix_body(scal_ref, x_ref, e_ref, mm_ref):
    i = pl.program_id(0)
    before = scal_ref[0]
    after = scal_ref[1]
    x = x_ref[...]
    new = jnp.where(x == before, after, x)
    rb, cb = _fix_idx(i)
    row = rb * 8 + jax.lax.broadcasted_iota(jnp.int32, (8, _FBW), 0)
    col = cb * _FBW + jax.lax.broadcasted_iota(jnp.int32, (8, _FBW), 1)
    keep = ((i < _FCB) & (row == 24)) | ((i >= _FCB) & (col >= _NSC))
    d = (new != e_ref[...]) & keep & (col < _N) & (row < _L)
    mismatch = jnp.any(d).astype(jnp.int32)

    @pl.when(i == 0)
    def _init():
        mm_ref[0] = mismatch

    @pl.when(i > 0)
    def _acc():
        mm_ref[0] = mm_ref[0] | mismatch


def _fix_reduce(xt, et, scal):
    return pl.pallas_call(
        _fix_body,
        grid=(_FG,),
        in_specs=[
            pl.BlockSpec(memory_space=pltpu.SMEM),
            pl.BlockSpec((8, _FBW), _fix_idx),
            pl.BlockSpec((8, _FBW), _fix_idx),
        ],
        out_specs=pl.BlockSpec(memory_space=pltpu.SMEM),
        out_shape=jax.ShapeDtypeStruct((1,), jnp.int32),
    )(scal, xt, et)


def kernel(ids, end_ids, reward_base, before_id, after_id):
    bef = jnp.full((16,), before_id, jnp.int32)
    aft = jnp.full((16,), after_id, jnp.int32)
    scal = jnp.stack([jnp.asarray(before_id, jnp.int32),
                      jnp.asarray(after_id, jnp.int32)])
    xt = ids.T
    et = end_ids.T
    mm = _sc_call(xt, et, bef, aft)
    out = _tc_main(xt, scal)
    fmm = _fix_reduce(xt, et, scal)
    done = jnp.logical_not(jnp.any(mm)) & (fmm[0] == 0)
    reward = jnp.where(done, reward_base[0], jnp.zeros((), jnp.float32))
    return out.T, done, reward


# speedup vs baseline: 1.7499x; 1.0035x over previous
import functools

import jax
import jax.numpy as jnp
from jax import lax
from jax.experimental import pallas as pl
from jax.experimental.pallas import tpu as pltpu
from jax.experimental.pallas import tpu_sc as plsc

_N, _L = 200000, 25
_NW = 32
_CH = 2432
_NSC = 199936
_NFULL = _NSC // _CH
_TAIL = _NSC - _NFULL * _CH
_NCHUNK = _NFULL + 1
_KPW = -(-_NCHUNK // _NW)
_ITEMS = [(s, k) for s in range(3) for k in range(_KPW)]

_mesh = plsc.VectorSubcoreMesh(core_axis_name="c", subcore_axis_name="s")


def _sc_body(xt, et, bef, aft, mm, xv, ev, befv, aftv, accv, sx, se):
    wid = lax.axis_index("s") * 2 + lax.axis_index("c")
    pltpu.sync_copy(bef, befv)
    pltpu.sync_copy(aft, aftv)

    def issue_in(s, k, b):
        c = wid + k * _NW
        r0 = s * 8
        cb = c * _CH

        @pl.when(c < _NFULL)
        def _main():
            pltpu.async_copy(xt.at[pl.ds(r0, 8), pl.ds(cb, _CH)],
                             xv.at[b], sx.at[b])
            pltpu.async_copy(et.at[pl.ds(r0, 8), pl.ds(cb, _CH)],
                             ev.at[b], se.at[b])

        @pl.when(c == _NFULL)
        def _tail():
            pltpu.async_copy(xt.at[pl.ds(r0, 8), pl.ds(cb, _TAIL)],
                             xv.at[b, :, pl.ds(0, _TAIL)], sx.at[b])
            pltpu.async_copy(et.at[pl.ds(r0, 8), pl.ds(cb, _TAIL)],
                             ev.at[b, :, pl.ds(0, _TAIL)], se.at[b])

    def wait_in(s, k, b):
        c = wid + k * _NW
        r0 = s * 8
        cb = c * _CH

        @pl.when(c < _NFULL)
        def _main():
            pltpu.make_async_copy(xt.at[pl.ds(r0, 8), pl.ds(cb, _CH)],
                                  xv.at[b], sx.at[b]).wait()
            pltpu.make_async_copy(et.at[pl.ds(r0, 8), pl.ds(cb, _CH)],
                                  ev.at[b], se.at[b]).wait()

        @pl.when(c == _NFULL)
        def _tail():
            pltpu.make_async_copy(xt.at[pl.ds(r0, 8), pl.ds(cb, _TAIL)],
                                  xv.at[b, :, pl.ds(0, _TAIL)], sx.at[b]).wait()
            pltpu.make_async_copy(et.at[pl.ds(r0, 8), pl.ds(cb, _TAIL)],
                                  ev.at[b, :, pl.ds(0, _TAIL)], se.at[b]).wait()

    def compute(s, k, b):
        c = wid + k * _NW

        def body(j, acc):
            col = j * 16
            for r in range(8):
                x = xv[b, r, pl.ds(col, 16)]
                e = ev[b, r, pl.ds(col, 16)]
                new = jnp.where(x == befv[...], aftv[...], x)
                acc = acc | (new ^ e)
            return acc

        @pl.when(c < _NFULL)
        def _main():
            accv[...] = lax.fori_loop(0, _CH // 16, body, accv[...],
                                      unroll=2)

        @pl.when(c == _NFULL)
        def _tail():
            accv[...] = lax.fori_loop(0, _TAIL // 16, body, accv[...],
                                      unroll=2)

    accv[...] = jnp.zeros((16,), jnp.int32)
    nit = len(_ITEMS)
    issue_in(*_ITEMS[0], 0)
    issue_in(*_ITEMS[1], 1)
    for idx, (s, k) in enumerate(_ITEMS):
        b = idx % 3
        c = wid + k * _NW
        if idx + 2 < nit:
            issue_in(*_ITEMS[idx + 2], (idx + 2) % 3)

        @pl.when(c < _NCHUNK)
        def _do(s=s, k=k, b=b):
            wait_in(s, k, b)
            compute(s, k, b)

    pltpu.sync_copy(accv, mm.at[wid])


_sc_call = functools.partial(
    pl.kernel,
    mesh=_mesh,
    out_type=jax.ShapeDtypeStruct((_NW, 16), jnp.int32),
    scratch_types=[
        pltpu.VMEM((3, 8, _CH), jnp.int32),
        pltpu.VMEM((3, 8, _CH), jnp.int32),
        pltpu.VMEM((16,), jnp.int32),
        pltpu.VMEM((16,), jnp.int32),
        pltpu.VMEM((16,), jnp.int32),
        pltpu.SemaphoreType.DMA((3,)),
        pltpu.SemaphoreType.DMA((3,)),
    ],
    compiler_params=pltpu.CompilerParams(use_tc_tiling_on_sc=True),
)(_sc_body)


_BW = 33408
_GRID = (_N + _BW - 1) // _BW


def _tc_body(scal_ref, x_ref, out_ref):
    before = scal_ref[0]
    after = scal_ref[1]
    x = x_ref[...]
    out_ref[...] = jnp.where(x == before, after, x)


def _tc_main(xt, scal):
    return pl.pallas_call(
        _tc_body,
        grid=(_GRID,),
        in_specs=[
            pl.BlockSpec(memory_space=pltpu.SMEM),
            pl.BlockSpec((_L, _BW), lambda i: (0, i)),
        ],
        out_specs=pl.BlockSpec((_L, _BW), lambda i: (0, i)),
        out_shape=jax.ShapeDtypeStruct((_L, _N), jnp.int32),
    )(scal, xt)


_FBW = 50176
_FCB = -(-_N // _FBW)
_FG = _FCB + 3


def _fix_idx(i):
    return (jnp.where(i < _FCB, 3, i - _FCB),
            jnp.where(i < _FCB, i, _FCB - 1))


def _fix_body(scal_ref, x_ref, e_ref, mm_ref):
    i = pl.program_id(0)
    before = scal_ref[0]
    after = scal_ref[1]
    x = x_ref[...]
    new = jnp.where(x == before, after, x)
    rb, cb = _fix_idx(i)
    row = rb * 8 + jax.lax.broadcasted_iota(jnp.int32, (8, _FBW), 0)
    col = cb * _FBW + jax.lax.broadcasted_iota(jnp.int32, (8, _FBW), 1)
    keep = ((i < _FCB) & (row == 24)) | ((i >= _FCB) & (col >= _NSC))
    d = (new != e_ref[...]) & keep & (col < _N) & (row < _L)
    mismatch = jnp.any(d).astype(jnp.int32)

    @pl.when(i == 0)
    def _init():
        mm_ref[0] = mismatch

    @pl.when(i > 0)
    def _acc():
        mm_ref[0] = mm_ref[0] | mismatch


def _fix_reduce(xt, et, scal):
    return pl.pallas_call(
        _fix_body,
        grid=(_FG,),
        in_specs=[
            pl.BlockSpec(memory_space=pltpu.SMEM),
            pl.BlockSpec((8, _FBW), _fix_idx),
            pl.BlockSpec((8, _FBW), _fix_idx),
        ],
        out_specs=pl.BlockSpec(memory_space=pltpu.SMEM),
        out_shape=jax.ShapeDtypeStruct((1,), jnp.int32),
    )(scal, xt, et)


def kernel(ids, end_ids, reward_base, before_id, after_id):
    bef = jnp.full((16,), before_id, jnp.int32)
    aft = jnp.full((16,), after_id, jnp.int32)
    scal = jnp.stack([jnp.asarray(before_id, jnp.int32),
                      jnp.asarray(after_id, jnp.int32)])
    xt = ids.T
    et = end_ids.T
    mm = _sc_call(xt, et, bef, aft)
    out = _tc_main(xt, scal)
    fmm = _fix_reduce(xt, et, scal)
    done = jnp.logical_not(jnp.any(mm)) & (fmm[0] == 0)
    reward = jnp.where(done, reward_base[0], jnp.zeros((), jnp.float32))
    return out.T, done, reward
